# baseline (speedup 1.0000x reference)
import jax
import jax.numpy as jnp
from jax.experimental import pallas as pl


def _copy_body(x_ref, o_ref):
    o_ref[...] = x_ref[...]


def _pallas_copy(x, rows_per_block):
    r, c = x.shape
    assert r % rows_per_block == 0, (r, rows_per_block)
    grid = r // rows_per_block
    return pl.pallas_call(
        _copy_body,
        grid=(grid,),
        in_specs=[pl.BlockSpec((rows_per_block, c), lambda i: (i, 0))],
        out_specs=pl.BlockSpec((rows_per_block, c), lambda i: (i, 0)),
        out_shape=jax.ShapeDtypeStruct((r, c), x.dtype),
    )(x)


def kernel(semantic, distance, vertex, embedding, direction):
    sem = _pallas_copy(semantic.reshape(128, 80000), 8).reshape(semantic.shape)
    dis = _pallas_copy(distance.reshape(96, 80000), 8).reshape(distance.shape)
    ver = _pallas_copy(vertex.reshape(2080, 1250), 208).reshape(vertex.shape)
    emb = _pallas_copy(embedding.reshape(512, 80000), 8).reshape(embedding.shape)
    dir_ = _pallas_copy(direction.reshape(1184, 80000), 8).reshape(direction.shape)
    return (sem, dis, ver, emb, dir_)
